# baseline (speedup 1.0000x reference)
import functools

import jax
import jax.numpy as jnp
from jax import lax
from jax.experimental import pallas as pl
from jax.experimental.pallas import tpu as pltpu
from jax.experimental.pallas import tpu_sc as plsc

_B, _C, _H, _W = 4, 96, 224, 224
_L = _H * _W
_E = 8
_TL = 3584
_NL = _L // _TL


def _prep_kernel(x_ref, xt_ref, pool_ref):
    blk = x_ref[0]
    xt_ref[0] = jnp.swapaxes(blk, 1, 2).astype(jnp.bfloat16)
    pool_ref[...] = jnp.sum(blk, axis=(1, 2))[None, :, None] * (1.0 / _L)


def _dot(a, b):
    return jax.lax.dot_general(a, b, (((1,), (0,)), ((), ())),
                               preferred_element_type=jnp.float32)


def _moe_kernel(ew_ref, x0_ref, xt_ref, x2_ref, x3_ref, we_ref, bet_ref,
                ewt_ref, jrev_ref, out_ref):
    b = pl.program_id(0)
    s = pl.program_id(2)
    m = [(ew_ref[b, d] * we_ref[d] +
          ew_ref[b, d + 4] * we_ref[d + 4]).astype(jnp.bfloat16)
         for d in range(4)]
    bias = _dot(bet_ref[...], ewt_ref[0])
    jrev = jrev_ref[...]
    nck = _TL // 128

    def emit(a, bt, c, d):
        out_ref[0] = (a.astype(jnp.float32) + _dot(m[0], a) + _dot(m[1], bt)
                      + bias)
        z = (_dot(m[2], c) + _dot(m[3], d)).astype(jnp.bfloat16)
        for k in range(nck):
            lo = (nck - 1 - k) * 128
            out_ref[0, :, lo:lo + 128] += _dot(z[:, k * 128:(k + 1) * 128],
                                               jrev)

    @pl.when(s == 0)
    def _():
        emit(x0_ref[0], xt_ref[0], x2_ref[0], x3_ref[0])

    @pl.when(s == 1)
    def _():
        emit(x2_ref[0], x3_ref[0], x0_ref[0], xt_ref[0])


def _gate_sc_body(pooled_hbm, wgt_hbm, bgp_hbm, ew_hbm, wgt_v, bg_v,
                  pooled_v, row_v):
    cid = lax.axis_index("c")
    sid = lax.axis_index("s")
    b = (sid * 2 + cid) % _B
    pltpu.sync_copy(pooled_hbm.at[b], pooled_v)
    pltpu.sync_copy(wgt_hbm, wgt_v)
    pltpu.sync_copy(bgp_hbm, bg_v)
    logits = bg_v[...]
    for cc in range(_C // 16):
        vec = pooled_v[pl.ds(cc * 16, 16)]
        for i in range(16):
            logits = logits + vec[i] * wgt_v[cc * 16 + i, :]
    lanes = lax.iota(jnp.int32, 16)
    masked = jnp.where(lanes < _E, logits, jnp.float32(-1e30))
    lv = [masked[i] for i in range(_E)]
    m1 = lv[0]
    i1 = jnp.int32(0)
    for e in range(1, _E):
        cond = lv[e] > m1
        m1 = jnp.where(cond, lv[e], m1)
        i1 = jnp.where(cond, jnp.int32(e), i1)
    m2 = jnp.float32(-3e30)
    i2 = jnp.int32(0)
    for e in range(_E):
        cond = jnp.logical_and(i1 != e, lv[e] > m2)
        m2 = jnp.where(cond, lv[e], m2)
        i2 = jnp.where(cond, jnp.int32(e), i2)
    es = jnp.exp(masked - m1)
    s = es[0]
    for i in range(1, _E):
        s = s + es[i]
    w = es / s
    keep = jnp.logical_or(lanes == i1, lanes == i2)
    row_v[...] = w * jnp.where(keep, jnp.float32(1.0), jnp.float32(0.0))

    @pl.when(sid < 2)
    def _():
        pltpu.sync_copy(row_v, ew_hbm.at[b])


def _gate_sc(pooled, wgt_pad, bg_pad):
    fn = functools.partial(
        pl.kernel,
        mesh=plsc.VectorSubcoreMesh(core_axis_name="c",
                                    subcore_axis_name="s"),
        out_type=jax.ShapeDtypeStruct((_B, 16), jnp.float32),
        scratch_types=[
            pltpu.VMEM((_C, 16), jnp.float32),
            pltpu.VMEM((16,), jnp.float32),
            pltpu.VMEM((_C,), jnp.float32),
            pltpu.VMEM((16,), jnp.float32),
        ],
    )(_gate_sc_body)
    return fn(pooled, wgt_pad, bg_pad)


@jax.jit
def kernel(inputs, Wg, bg, We, be):
    jrev = jnp.flip(jnp.eye(128, dtype=jnp.bfloat16), 1)

    _CB = 32
    xT4, pooled = pl.pallas_call(
        _prep_kernel,
        grid=(_B, _C // _CB),
        in_specs=[pl.BlockSpec((1, _CB, _H, _W), lambda b, c: (b, c, 0, 0))],
        out_specs=[
            pl.BlockSpec((1, _CB, _W, _H), lambda b, c: (b, c, 0, 0)),
            pl.BlockSpec((1, _CB, 1), lambda b, c: (b, c, 0)),
        ],
        out_shape=[
            jax.ShapeDtypeStruct((_B, _C, _W, _H), jnp.bfloat16),
            jax.ShapeDtypeStruct((_B, _C, 1), jnp.float32),
        ],
        compiler_params=pltpu.CompilerParams(
            dimension_semantics=("arbitrary", "arbitrary")),
    )(inputs)
    x = inputs.astype(jnp.bfloat16).reshape(_B, _C, _L)
    xT = xT4.reshape(_B, _C, _L)
    pooled = pooled[:, :, 0]

    wgt_pad = jnp.zeros((_C, 16), jnp.float32).at[:, :_E].set(Wg.T)
    bg_pad = jnp.zeros((16,), jnp.float32).at[:_E].set(bg)
    ew = _gate_sc(pooled, wgt_pad, bg_pad)[:, :_E]
    ewT = ew.reshape(_B, _E, 1)
    beT = jnp.swapaxes(be, 0, 1)

    out = pl.pallas_call(
        _moe_kernel,
        grid=(_B, _NL // 2, 2),
        in_specs=[
            pl.BlockSpec(memory_space=pltpu.SMEM),
            pl.BlockSpec((1, _C, _TL), lambda b, j, s: (b, 0, j)),
            pl.BlockSpec((1, _C, _TL), lambda b, j, s: (b, 0, j)),
            pl.BlockSpec((1, _C, _TL),
                         lambda b, j, s: (b, 0, _NL - 1 - j)),
            pl.BlockSpec((1, _C, _TL),
                         lambda b, j, s: (b, 0, _NL - 1 - j)),
            pl.BlockSpec((_E, _C, _C), lambda b, j, s: (0, 0, 0)),
            pl.BlockSpec((_C, _E), lambda b, j, s: (0, 0)),
            pl.BlockSpec((1, _E, 1), lambda b, j, s: (b, 0, 0)),
            pl.BlockSpec((128, 128), lambda b, j, s: (0, 0)),
        ],
        out_specs=pl.BlockSpec(
            (1, _C, _TL),
            lambda b, j, s: (b, 0, j + s * (_NL - 1 - 2 * j))),
        out_shape=jax.ShapeDtypeStruct((_B, _C, _L), jnp.float32),
        compiler_params=pltpu.CompilerParams(
            dimension_semantics=("parallel", "parallel", "arbitrary")),
    )(ew, x, xT, x, xT, We, beT, ewT, jrev)

    return out.reshape(_B, _C, _H, _W)


# speedup vs baseline: 1.0711x; 1.0711x over previous
import functools

import jax
import jax.numpy as jnp
from jax import lax
from jax.experimental import pallas as pl
from jax.experimental.pallas import tpu as pltpu
from jax.experimental.pallas import tpu_sc as plsc

_B, _C, _H, _W = 4, 96, 224, 224
_L = _H * _W
_E = 8
_TL = 3584
_NL = _L // _TL


def _prep_kernel(x_ref, xb_ref, xt_ref, pool_ref):
    blk = x_ref[0]
    xb_ref[0] = blk.astype(jnp.bfloat16)
    xt_ref[0] = jnp.swapaxes(blk, 1, 2).astype(jnp.bfloat16)
    pool_ref[...] = jnp.sum(blk, axis=(1, 2))[None, :, None] * (1.0 / _L)


def _dot(a, b):
    return jax.lax.dot_general(a, b, (((1,), (0,)), ((), ())),
                               preferred_element_type=jnp.float32)


def _moe_kernel(ew_ref, x0_ref, xt_ref, x2_ref, x3_ref, we_ref, bet_ref,
                ewt_ref, jrev_ref, out_ref):
    b = pl.program_id(0)
    s = pl.program_id(2)
    m = [(ew_ref[b, d] * we_ref[d] +
          ew_ref[b, d + 4] * we_ref[d + 4]).astype(jnp.bfloat16)
         for d in range(4)]
    bias = _dot(bet_ref[...], ewt_ref[0])
    jrev = jrev_ref[...]
    nck = _TL // 128

    def emit(a, bt, c, d):
        out_ref[0] = (a.astype(jnp.float32) + _dot(m[0], a) + _dot(m[1], bt)
                      + bias)
        z = (_dot(m[2], c) + _dot(m[3], d)).astype(jnp.bfloat16)
        for k in range(nck):
            lo = (nck - 1 - k) * 128
            out_ref[0, :, lo:lo + 128] += _dot(z[:, k * 128:(k + 1) * 128],
                                               jrev)

    @pl.when(s == 0)
    def _():
        emit(x0_ref[0], xt_ref[0], x2_ref[0], x3_ref[0])

    @pl.when(s == 1)
    def _():
        emit(x2_ref[0], x3_ref[0], x0_ref[0], xt_ref[0])


def _gate_sc_body(pooled_hbm, wgt_hbm, bgp_hbm, ew_hbm, wgt_v, bg_v,
                  pooled_v, row_v):
    cid = lax.axis_index("c")
    sid = lax.axis_index("s")
    b = (sid * 2 + cid) % _B
    pltpu.sync_copy(pooled_hbm.at[b], pooled_v)
    pltpu.sync_copy(wgt_hbm, wgt_v)
    pltpu.sync_copy(bgp_hbm, bg_v)
    logits = bg_v[...]
    for cc in range(_C // 16):
        vec = pooled_v[pl.ds(cc * 16, 16)]
        for i in range(16):
            logits = logits + vec[i] * wgt_v[cc * 16 + i, :]
    lanes = lax.iota(jnp.int32, 16)
    masked = jnp.where(lanes < _E, logits, jnp.float32(-1e30))
    lv = [masked[i] for i in range(_E)]
    m1 = lv[0]
    i1 = jnp.int32(0)
    for e in range(1, _E):
        cond = lv[e] > m1
        m1 = jnp.where(cond, lv[e], m1)
        i1 = jnp.where(cond, jnp.int32(e), i1)
    m2 = jnp.float32(-3e30)
    i2 = jnp.int32(0)
    for e in range(_E):
        cond = jnp.logical_and(i1 != e, lv[e] > m2)
        m2 = jnp.where(cond, lv[e], m2)
        i2 = jnp.where(cond, jnp.int32(e), i2)
    es = jnp.exp(masked - m1)
    s = es[0]
    for i in range(1, _E):
        s = s + es[i]
    w = es / s
    keep = jnp.logical_or(lanes == i1, lanes == i2)
    row_v[...] = w * jnp.where(keep, jnp.float32(1.0), jnp.float32(0.0))

    @pl.when(sid < 2)
    def _():
        pltpu.sync_copy(row_v, ew_hbm.at[b])


def _gate_sc(pooled, wgt_pad, bg_pad):
    fn = functools.partial(
        pl.kernel,
        mesh=plsc.VectorSubcoreMesh(core_axis_name="c",
                                    subcore_axis_name="s"),
        out_type=jax.ShapeDtypeStruct((_B, 16), jnp.float32),
        scratch_types=[
            pltpu.VMEM((_C, 16), jnp.float32),
            pltpu.VMEM((16,), jnp.float32),
            pltpu.VMEM((_C,), jnp.float32),
            pltpu.VMEM((16,), jnp.float32),
        ],
    )(_gate_sc_body)
    return fn(pooled, wgt_pad, bg_pad)


@jax.jit
def kernel(inputs, Wg, bg, We, be):
    jrev = jnp.flip(jnp.eye(128, dtype=jnp.bfloat16), 1)

    _CB = 48
    xb4, xT4, pooled = pl.pallas_call(
        _prep_kernel,
        grid=(_B, _C // _CB),
        in_specs=[pl.BlockSpec((1, _CB, _H, _W), lambda b, c: (b, c, 0, 0))],
        out_specs=[
            pl.BlockSpec((1, _CB, _H, _W), lambda b, c: (b, c, 0, 0)),
            pl.BlockSpec((1, _CB, _W, _H), lambda b, c: (b, c, 0, 0)),
            pl.BlockSpec((1, _CB, 1), lambda b, c: (b, c, 0)),
        ],
        out_shape=[
            jax.ShapeDtypeStruct((_B, _C, _H, _W), jnp.bfloat16),
            jax.ShapeDtypeStruct((_B, _C, _W, _H), jnp.bfloat16),
            jax.ShapeDtypeStruct((_B, _C, 1), jnp.float32),
        ],
        compiler_params=pltpu.CompilerParams(
            dimension_semantics=("arbitrary", "arbitrary")),
    )(inputs)
    x = xb4.reshape(_B, _C, _L)
    xT = xT4.reshape(_B, _C, _L)
    pooled = pooled[:, :, 0]

    wgt_pad = jnp.zeros((_C, 16), jnp.float32).at[:, :_E].set(Wg.T)
    bg_pad = jnp.zeros((16,), jnp.float32).at[:_E].set(bg)
    ew = _gate_sc(pooled, wgt_pad, bg_pad)[:, :_E]
    ewT = ew.reshape(_B, _E, 1)
    beT = jnp.swapaxes(be, 0, 1)

    out = pl.pallas_call(
        _moe_kernel,
        grid=(_B, _NL // 2, 2),
        in_specs=[
            pl.BlockSpec(memory_space=pltpu.SMEM),
            pl.BlockSpec((1, _C, _TL), lambda b, j, s: (b, 0, j)),
            pl.BlockSpec((1, _C, _TL), lambda b, j, s: (b, 0, j)),
            pl.BlockSpec((1, _C, _TL),
                         lambda b, j, s: (b, 0, _NL - 1 - j)),
            pl.BlockSpec((1, _C, _TL),
                         lambda b, j, s: (b, 0, _NL - 1 - j)),
            pl.BlockSpec((_E, _C, _C), lambda b, j, s: (0, 0, 0)),
            pl.BlockSpec((_C, _E), lambda b, j, s: (0, 0)),
            pl.BlockSpec((1, _E, 1), lambda b, j, s: (b, 0, 0)),
            pl.BlockSpec((128, 128), lambda b, j, s: (0, 0)),
        ],
        out_specs=pl.BlockSpec(
            (1, _C, _TL),
            lambda b, j, s: (b, 0, j + s * (_NL - 1 - 2 * j))),
        out_shape=jax.ShapeDtypeStruct((_B, _C, _L), jnp.float32),
        compiler_params=pltpu.CompilerParams(
            dimension_semantics=("parallel", "parallel", "arbitrary")),
    )(ew, x, xT, x, xT, We, beT, ewT, jrev)

    return out.reshape(_B, _C, _H, _W)
